# baseline (speedup 1.0000x reference)
import functools

import numpy as np
import jax
import jax.numpy as jnp
from jax import lax
from jax.experimental import pallas as pl
from jax.experimental.pallas import tpu as pltpu
from jax.experimental.pallas import tpu_sc as plsc

_C, _H, _W = 3, 2048, 2048
_NUM_CRACKS = 100
_MAX_LENGTH = 2
_MAX_WIDTH = 2


def _scratch_mask_np(cols, rows, seed=0):
    rng = np.random.default_rng(seed)
    n = int(rng.integers(1, _NUM_CRACKS))
    x_start = rng.integers(0, rows, size=n)
    x_end = rng.integers(0, rows, size=n)
    y_start = rng.integers(0, cols, size=n)
    y_end = rng.integers(0, cols, size=n)
    length = rng.integers(1, _MAX_LENGTH, size=n)
    width = rng.integers(1, _MAX_WIDTH, size=n)
    mask = np.zeros((cols, rows), dtype=bool)
    for i in range(n):
        xs, xe = int(x_start[i]), int(x_end[i])
        ys, ye = int(y_start[i]), int(y_end[i])
        l, w = int(length[i]), int(width[i])
        dx, dy = abs(xe - xs), abs(ye - ys)
        sx = 1 if xs < xe else -1
        sy = 1 if ys < ye else -1
        err = dx - dy
        while xs != xe or ys != ye:
            mask[ys:ys + w, xs:xs + l] = True
            e2 = 2 * err
            if e2 > -dy:
                err -= dy
                xs += sx
            if e2 < dx:
                err += dx
                ys += sy
    return mask


_NW = 32
_CHUNK = 128

_pix = np.flatnonzero(_scratch_mask_np(_H, _W))
_flat = (_pix[None, :] + (np.arange(_C) * _H * _W)[:, None]).reshape(-1)
_KCH = -(-_flat.size // (_NW * _CHUNK))
_pad = _NW * _CHUNK * _KCH - _flat.size
_flat = np.concatenate([_flat, np.full(_pad, _flat[-1], dtype=_flat.dtype)])
_IDX_NP = _flat.astype(np.int32).reshape(_NW, _KCH, _CHUNK)


_ROWS = _C * _H
_BLK = 512


def _copy_max_body(x_ref, o_ref, m_ref):
    o_ref[...] = x_ref[...]
    bm = jnp.max(x_ref[...])

    @pl.when(pl.program_id(0) == 0)
    def _():
        m_ref[0, 0] = bm

    @pl.when(pl.program_id(0) != 0)
    def _():
        m_ref[0, 0] = jnp.maximum(m_ref[0, 0], bm)


_copy_max = pl.pallas_call(
    _copy_max_body,
    grid=(_ROWS // _BLK,),
    in_specs=[pl.BlockSpec((_BLK, _W), lambda i: (i, 0))],
    out_specs=[
        pl.BlockSpec((_BLK, _W), lambda i: (i, 0)),
        pl.BlockSpec(memory_space=pltpu.SMEM),
    ],
    out_shape=[
        jax.ShapeDtypeStruct((_ROWS, _W), jnp.float32),
        jax.ShapeDtypeStruct((1, 1), jnp.float32),
    ],
)


@functools.cache
def _get_scatter_kernel():
    mesh = plsc.VectorSubcoreMesh(core_axis_name="c", subcore_axis_name="s")
    num_cores = mesh.num_cores

    @functools.partial(
        pl.kernel,
        out_type=(),
        mesh=mesh,
        scratch_types=[
            pltpu.VMEM((_KCH, _CHUNK), jnp.int32),
            pltpu.VMEM((_CHUNK,), jnp.float32),
            pltpu.VMEM((16,), jnp.float32),
            pltpu.SemaphoreType.DMA,
        ],
    )
    def _scatter_kernel(img_ref, idx_hbm, val_hbm, idx_v, vals_v, val_v, sem):
        wid = lax.axis_index("s") * num_cores + lax.axis_index("c")
        pltpu.sync_copy(idx_hbm.at[wid], idx_v)
        pltpu.sync_copy(val_hbm, val_v)
        v = val_v[...]
        for i in range(_CHUNK // 16):
            vals_v[pl.ds(16 * i, 16)] = v

        @pl.loop(0, _KCH)
        def _(j):
            pltpu.make_async_copy(vals_v, img_ref.at[idx_v.at[j]], sem).start()

        @pl.loop(0, _KCH)
        def _(j):
            pltpu.make_async_copy(vals_v, img_ref.at[idx_v.at[j]], sem).wait()

    return _scatter_kernel


def kernel(inp):
    img, val = _copy_max(inp.reshape(_ROWS, _W))
    val16 = jnp.broadcast_to(val.reshape(1), (16,))
    img_ref = jax.new_ref(img.reshape(-1))
    _get_scatter_kernel()(img_ref, jnp.asarray(_IDX_NP), val16)
    return img_ref[...].reshape(_C, _H, _W)


# speedup vs baseline: 1.7064x; 1.7064x over previous
import functools

import numpy as np
import jax
import jax.numpy as jnp
from jax import lax
from jax.experimental import pallas as pl
from jax.experimental.pallas import tpu as pltpu
from jax.experimental.pallas import tpu_sc as plsc

_C, _H, _W = 3, 2048, 2048
_NUM_CRACKS = 100
_MAX_LENGTH = 2
_MAX_WIDTH = 2


def _scratch_mask_np(cols, rows, seed=0):
    rng = np.random.default_rng(seed)
    n = int(rng.integers(1, _NUM_CRACKS))
    x_start = rng.integers(0, rows, size=n)
    x_end = rng.integers(0, rows, size=n)
    y_start = rng.integers(0, cols, size=n)
    y_end = rng.integers(0, cols, size=n)
    length = rng.integers(1, _MAX_LENGTH, size=n)
    width = rng.integers(1, _MAX_WIDTH, size=n)
    mask = np.zeros((cols, rows), dtype=bool)
    for i in range(n):
        xs, xe = int(x_start[i]), int(x_end[i])
        ys, ye = int(y_start[i]), int(y_end[i])
        l, w = int(length[i]), int(width[i])
        dx, dy = abs(xe - xs), abs(ye - ys)
        sx = 1 if xs < xe else -1
        sy = 1 if ys < ye else -1
        err = dx - dy
        while xs != xe or ys != ye:
            mask[ys:ys + w, xs:xs + l] = True
            e2 = 2 * err
            if e2 > -dy:
                err -= dy
                xs += sx
            if e2 < dx:
                err += dx
                ys += sy
    return mask


_NW = 32
_CHUNK = 128

_pix = np.flatnonzero(_scratch_mask_np(_H, _W))
_flat = (_pix[None, :] + (np.arange(_C) * _H * _W)[:, None]).reshape(-1)
_KCH = -(-_flat.size // (_NW * _CHUNK))
_pad = _NW * _CHUNK * _KCH - _flat.size
_flat = np.concatenate([_flat, _flat[:_pad]])
_IDX_NP = _flat.astype(np.int32).reshape(_NW, _KCH * _CHUNK)


_ROWS = _C * _H
_BLK = 512


def _copy_max_body(x_ref, o_ref, m_ref):
    o_ref[...] = x_ref[...]
    bm = jnp.max(x_ref[...])

    @pl.when(pl.program_id(0) == 0)
    def _():
        m_ref[0, 0] = bm

    @pl.when(pl.program_id(0) != 0)
    def _():
        m_ref[0, 0] = jnp.maximum(m_ref[0, 0], bm)


_copy_max = pl.pallas_call(
    _copy_max_body,
    grid=(_ROWS // _BLK,),
    in_specs=[pl.BlockSpec((_BLK, _W), lambda i: (i, 0))],
    out_specs=[
        pl.BlockSpec((_BLK, _W), lambda i: (i, 0)),
        pl.BlockSpec(memory_space=pltpu.SMEM),
    ],
    out_shape=[
        jax.ShapeDtypeStruct((_ROWS, _W), jnp.float32),
        jax.ShapeDtypeStruct((1, 1), jnp.float32),
    ],
)


@functools.cache
def _get_scatter_kernel():
    mesh = plsc.VectorSubcoreMesh(core_axis_name="c", subcore_axis_name="s")
    num_cores = mesh.num_cores

    @functools.partial(
        pl.kernel,
        out_type=(),
        mesh=mesh,
        scratch_types=[
            pltpu.VMEM((_KCH * _CHUNK,), jnp.int32),
            pltpu.VMEM((_KCH * _CHUNK,), jnp.float32),
            pltpu.VMEM((16,), jnp.float32),
            pltpu.SemaphoreType.DMA,
        ],
    )
    def _scatter_kernel(img_ref, idx_hbm, val_hbm, idx_v, vals_v, val_v, sem):
        wid = lax.axis_index("s") * num_cores + lax.axis_index("c")
        pltpu.sync_copy(idx_hbm.at[wid], idx_v)
        pltpu.sync_copy(val_hbm, val_v)
        v = val_v[...]

        @pl.loop(0, _KCH * _CHUNK // 16)
        def _(i):
            vals_v[pl.ds(16 * i, 16)] = v

        pltpu.async_copy(vals_v, img_ref.at[idx_v], sem).wait()

    return _scatter_kernel


def kernel(inp):
    img, val = _copy_max(inp.reshape(_ROWS, _W))
    val16 = jnp.broadcast_to(val.reshape(1), (16,))
    img_ref = jax.new_ref(img.reshape(-1))
    _get_scatter_kernel()(img_ref, jnp.asarray(_IDX_NP), val16)
    return img_ref[...].reshape(_C, _H, _W)


# speedup vs baseline: 23.4777x; 13.7589x over previous
import numpy as np
import jax
import jax.numpy as jnp
from jax import lax
from jax.experimental import pallas as pl
from jax.experimental.pallas import tpu as pltpu

_C, _H, _W = 3, 2048, 2048
_NUM_CRACKS = 100
_MAX_LENGTH = 2
_MAX_WIDTH = 2


def _scratch_mask_np(cols, rows, seed=0):
    rng = np.random.default_rng(seed)
    n = int(rng.integers(1, _NUM_CRACKS))
    x_start = rng.integers(0, rows, size=n)
    x_end = rng.integers(0, rows, size=n)
    y_start = rng.integers(0, cols, size=n)
    y_end = rng.integers(0, cols, size=n)
    length = rng.integers(1, _MAX_LENGTH, size=n)
    width = rng.integers(1, _MAX_WIDTH, size=n)
    mask = np.zeros((cols, rows), dtype=bool)
    for i in range(n):
        xs, xe = int(x_start[i]), int(x_end[i])
        ys, ye = int(y_start[i]), int(y_end[i])
        l, w = int(length[i]), int(width[i])
        dx, dy = abs(xe - xs), abs(ye - ys)
        sx = 1 if xs < xe else -1
        sy = 1 if ys < ye else -1
        err = dx - dy
        while xs != xe or ys != ye:
            mask[ys:ys + w, xs:xs + l] = True
            e2 = 2 * err
            if e2 > -dy:
                err -= dy
                xs += sx
            if e2 < dx:
                err += dx
                ys += sy
    return mask


_MASK_NP = _scratch_mask_np(_H, _W).astype(np.int8)

_ROWS = _C * _H
_NB = 24
_BR = _ROWS // _NB
_BPC = _H // _BR


def _body(x_hbm, mask_hbm, o_hbm, img_v, mask_v, sem_in, sem_out, sem_msk):
    pltpu.make_async_copy(mask_hbm, mask_v, sem_msk).start()
    for b in range(_NB):
        pltpu.make_async_copy(
            x_hbm.at[pl.ds(b * _BR, _BR), :],
            img_v.at[pl.ds(b * _BR, _BR), :],
            sem_in.at[b],
        ).start()

    def _reduce(b, m):
        pltpu.make_async_copy(
            x_hbm.at[pl.ds(b * _BR, _BR), :],
            img_v.at[pl.ds(b * _BR, _BR), :],
            sem_in.at[b],
        ).wait()
        return jnp.maximum(m, jnp.max(img_v[pl.ds(b * _BR, _BR), :]))

    val = lax.fori_loop(0, _NB, _reduce, -jnp.inf, unroll=True)

    pltpu.make_async_copy(mask_hbm, mask_v, sem_msk).wait()

    def _writeback(b, _):
        r = lax.rem(b, _BPC) * _BR
        mb = mask_v[pl.ds(r, _BR), :] != 0
        img_v[pl.ds(b * _BR, _BR), :] = jnp.where(
            mb, val, img_v[pl.ds(b * _BR, _BR), :]
        )
        pltpu.make_async_copy(
            img_v.at[pl.ds(b * _BR, _BR), :],
            o_hbm.at[pl.ds(b * _BR, _BR), :],
            sem_out.at[b],
        ).start()
        return 0

    lax.fori_loop(0, _NB, _writeback, 0, unroll=True)

    def _drain(b, _):
        pltpu.make_async_copy(
            img_v.at[pl.ds(b * _BR, _BR), :],
            o_hbm.at[pl.ds(b * _BR, _BR), :],
            sem_out.at[b],
        ).wait()
        return 0

    lax.fori_loop(0, _NB, _drain, 0, unroll=True)


_overlay = pl.pallas_call(
    _body,
    in_specs=[
        pl.BlockSpec(memory_space=pl.ANY),
        pl.BlockSpec(memory_space=pl.ANY),
    ],
    out_specs=pl.BlockSpec(memory_space=pl.ANY),
    out_shape=jax.ShapeDtypeStruct((_ROWS, _W), jnp.float32),
    scratch_shapes=[
        pltpu.VMEM((_ROWS, _W), jnp.float32),
        pltpu.VMEM((_H, _W), jnp.int8),
        pltpu.SemaphoreType.DMA((_NB,)),
        pltpu.SemaphoreType.DMA((_NB,)),
        pltpu.SemaphoreType.DMA,
    ],
    compiler_params=pltpu.CompilerParams(
        vmem_limit_bytes=64 * 1024 * 1024,
    ),
)


def kernel(inp):
    out = _overlay(inp.reshape(_ROWS, _W), jnp.asarray(_MASK_NP))
    return out.reshape(_C, _H, _W)
